# baseline (speedup 1.0000x reference)
import functools

import jax
import jax.numpy as jnp
from jax import lax
from jax.experimental import pallas as pl
from jax.experimental.pallas import tpu as pltpu
from jax.experimental.pallas import tpu_sc as plsc

DIM = 64
NC = 2
NS = 16
NW = NC * NS
NB = 4


def _gather_sc(pe, idx_flat, nbatch, seq):
    b_per_w = nbatch // NW
    n_iter = b_per_w // NB
    rows_per_iter = NB * seq
    mesh = plsc.VectorSubcoreMesh(core_axis_name="c", subcore_axis_name="s")

    @functools.partial(
        pl.kernel,
        mesh=mesh,
        out_type=jax.ShapeDtypeStruct((nbatch, seq, DIM), jnp.float32),
        scratch_types=[
            pltpu.VMEM((rows_per_iter,), jnp.int32),
            pltpu.VMEM((rows_per_iter, DIM), jnp.float32),
            pltpu.SemaphoreType.DMA,
        ],
        compiler_params=pltpu.CompilerParams(use_tc_tiling_on_sc=False),
    )
    def k(pe_hbm, idx_hbm, out_hbm, idx_v, rows_v, sem):
        wid = lax.axis_index("s") * NC + lax.axis_index("c")
        base = wid * b_per_w

        def body(i, carry):
            b0 = base + i * NB
            pltpu.sync_copy(idx_hbm.at[pl.ds(b0 * seq, rows_per_iter)], idx_v)
            pltpu.async_copy(pe_hbm.at[idx_v], rows_v, sem).wait()
            for kk in range(NB):
                pltpu.sync_copy(rows_v.at[pl.ds(kk * seq, seq)],
                                out_hbm.at[b0 + kk])
            return carry

        lax.fori_loop(0, n_iter, body, 0)

    return k(pe, idx_flat)


def kernel(pos, pe):
    b, l = pos.shape
    flat = pos.reshape(-1).astype(jnp.int32)
    return _gather_sc(pe, flat, b, l)


# speedup vs baseline: 1.3077x; 1.3077x over previous
import functools

import jax
import jax.numpy as jnp
from jax import lax
from jax.experimental import pallas as pl
from jax.experimental.pallas import tpu as pltpu
from jax.experimental.pallas import tpu_sc as plsc

DIM = 64
NC = 2
NS = 16
BBLK = 2048
QUART = 4096


def _gather_t(pe_t, pos_t, seq, nbatch, nrows):
    mesh = plsc.VectorSubcoreMesh(core_axis_name="c", subcore_axis_name="s")

    @functools.partial(
        pl.kernel,
        mesh=mesh,
        out_type=jax.ShapeDtypeStruct((seq, DIM, nbatch), jnp.float32),
        scratch_types=[
            pltpu.VMEM((8, nrows), jnp.float32),
            pltpu.VMEM((QUART,), jnp.int32),
            pltpu.VMEM((QUART,), jnp.int32),
            pltpu.VMEM((8, BBLK), jnp.float32),
            pltpu.VMEM((8, BBLK), jnp.float32),
            pltpu.SemaphoreType.DMA,
            pltpu.SemaphoreType.DMA,
            pltpu.SemaphoreType.DMA,
            pltpu.SemaphoreType.DMA,
        ],
        compiler_params=pltpu.CompilerParams(
            use_tc_tiling_on_sc=True, needs_layout_passes=False),
    )
    def k(pe_hbm, pos_hbm, out_hbm, pe_v, idx0, idx1, wb0, wb1,
          isem0, isem1, osem0, osem1):
        c = lax.axis_index("c")
        s = lax.axis_index("s")
        octet = lax.rem(s, 8)
        quarter = c * 2 + s // 8
        d0 = octet * 8
        bq = quarter * QUART
        idx_v = (idx0, idx1)
        isem = (isem0, isem1)
        wb = (wb0, wb1)
        osem = (osem0, osem1)

        pltpu.sync_copy(pe_hbm.at[pl.ds(d0, 8), :], pe_v)
        pltpu.async_copy(pos_hbm.at[0, pl.ds(bq, QUART)], idx0, isem0)

        def gather_block(src_idx, boff, dst):
            def g_body(g, carry):
                i16 = src_idx[pl.ds(boff + g * 16, 16)]
                for qd in range(8):
                    rows = jnp.full((16,), qd, jnp.int32)
                    dst[qd, pl.ds(g * 16, 16)] = plsc.load_gather(
                        pe_v, [rows, i16])
                return carry
            lax.fori_loop(0, BBLK // 16, g_body, 0)

        def body(l, carry):
            p = lax.rem(l, 2)
            for pp in range(2):
                @pl.when(p == pp)
                def _():
                    pltpu.make_async_copy(
                        pos_hbm.at[0, pl.ds(bq, QUART)],
                        idx_v[pp], isem[pp]).wait()

                    @pl.when(l + 1 < seq)
                    def _():
                        pltpu.async_copy(
                            pos_hbm.at[l + 1, pl.ds(bq, QUART)],
                            idx_v[1 - pp], isem[1 - pp])

                    for bb in range(2):
                        @pl.when(l >= 1)
                        def _():
                            pltpu.make_async_copy(
                                wb[bb],
                                out_hbm.at[0, pl.ds(d0, 8),
                                           pl.ds(bq + bb * BBLK, BBLK)],
                                osem[bb]).wait()
                        gather_block(idx_v[pp], bb * BBLK, wb[bb])
                        pltpu.async_copy(
                            wb[bb],
                            out_hbm.at[l, pl.ds(d0, 8),
                                       pl.ds(bq + bb * BBLK, BBLK)],
                            osem[bb])
            return carry

        lax.fori_loop(0, seq, body, 0)
        for bb in range(2):
            pltpu.make_async_copy(
                wb[bb],
                out_hbm.at[0, pl.ds(d0, 8), pl.ds(bq + bb * BBLK, BBLK)],
                osem[bb]).wait()

    return k(pe_t, pos_t)


def kernel(pos, pe):
    b, l = pos.shape
    pos_t = pos.T.astype(jnp.int32)
    pe_t = pe.T
    w = _gather_t(pe_t, pos_t, l, b, pe.shape[0])
    return jnp.transpose(w, (2, 0, 1))


# speedup vs baseline: 6.7960x; 5.1967x over previous
import functools

import jax
import jax.numpy as jnp
from jax import lax
from jax.experimental import pallas as pl
from jax.experimental.pallas import tpu as pltpu
from jax.experimental.pallas import tpu_sc as plsc

DIM = 64
NC = 2
NS = 16
BBLK = 2048
QUART = 4096


def _gather_t(pe_t, pos_t, seq, nbatch, nrows):
    mesh = plsc.VectorSubcoreMesh(core_axis_name="c", subcore_axis_name="s")

    @functools.partial(
        pl.kernel,
        mesh=mesh,
        out_type=jax.ShapeDtypeStruct((seq, DIM, nbatch), jnp.float32),
        scratch_types=[
            pltpu.VMEM((8, nrows), jnp.float32),
            pltpu.VMEM((QUART,), jnp.int32),
            pltpu.VMEM((QUART,), jnp.int32),
            pltpu.VMEM((8, BBLK), jnp.float32),
            pltpu.VMEM((8, BBLK), jnp.float32),
            pltpu.SemaphoreType.DMA,
            pltpu.SemaphoreType.DMA,
            pltpu.SemaphoreType.DMA,
            pltpu.SemaphoreType.DMA,
        ],
        compiler_params=pltpu.CompilerParams(
            use_tc_tiling_on_sc=True, needs_layout_passes=False),
    )
    def k(pe_hbm, pos_hbm, out_hbm, pe_v, idx0, idx1, wb0, wb1,
          isem0, isem1, osem0, osem1):
        c = lax.axis_index("c")
        s = lax.axis_index("s")
        octet = lax.rem(s, 8)
        quarter = c * 2 + s // 8
        d0 = octet * 8
        bq = quarter * QUART
        idx_v = (idx0, idx1)
        isem = (isem0, isem1)
        wb = (wb0, wb1)
        osem = (osem0, osem1)

        pltpu.sync_copy(pe_hbm.at[pl.ds(d0, 8), :], pe_v)
        pltpu.async_copy(pos_hbm.at[0, pl.ds(bq, QUART)], idx0, isem0)

        def gather_block(src_idx, boff, dst):
            @plsc.parallel_loop(0, BBLK // 16, 1, unroll=4)
            def g_body(g):
                i16 = src_idx[pl.ds(boff + g * 16, 16)]
                for qd in range(8):
                    rows = jnp.full((16,), qd, jnp.int32)
                    dst[qd, pl.ds(g * 16, 16)] = plsc.load_gather(
                        pe_v, [rows, i16])

        def body(l, carry):
            p = lax.rem(l, 2)
            for pp in range(2):
                @pl.when(p == pp)
                def _():
                    pltpu.make_async_copy(
                        pos_hbm.at[0, pl.ds(bq, QUART)],
                        idx_v[pp], isem[pp]).wait()

                    @pl.when(l + 1 < seq)
                    def _():
                        pltpu.async_copy(
                            pos_hbm.at[l + 1, pl.ds(bq, QUART)],
                            idx_v[1 - pp], isem[1 - pp])

                    for bb in range(2):
                        @pl.when(l >= 1)
                        def _():
                            pltpu.make_async_copy(
                                wb[bb],
                                out_hbm.at[0, pl.ds(d0, 8),
                                           pl.ds(bq + bb * BBLK, BBLK)],
                                osem[bb]).wait()
                        gather_block(idx_v[pp], bb * BBLK, wb[bb])
                        pltpu.async_copy(
                            wb[bb],
                            out_hbm.at[l, pl.ds(d0, 8),
                                       pl.ds(bq + bb * BBLK, BBLK)],
                            osem[bb])
            return carry

        lax.fori_loop(0, seq, body, 0)
        for bb in range(2):
            pltpu.make_async_copy(
                wb[bb],
                out_hbm.at[0, pl.ds(d0, 8), pl.ds(bq + bb * BBLK, BBLK)],
                osem[bb]).wait()

    return k(pe_t, pos_t)


def kernel(pos, pe):
    b, l = pos.shape
    pos_t = pos.T.astype(jnp.int32)
    pe_t = pe.T
    w = _gather_t(pe_t, pos_t, l, b, pe.shape[0])
    return jnp.transpose(w, (2, 0, 1))
